# baseline (speedup 1.0000x reference)
import jax
import jax.numpy as jnp
from jax import lax
from jax.experimental import pallas as pl
from jax.experimental.pallas import tpu as pltpu
from jax.experimental.pallas import tpu_sc as plsc

_N = 100000
_KNN = 16
_D0, _D1 = 128, 256
_H0, _H1 = 32, 16
_NCAT = 10
_ZW = 16

_BN = 4096
_NB = (_N + _BN - 1) // _BN
_NS_TOT = _NB * _BN
_PR = _NS_TOT // 8

_NC, _NSC = 2, 16
_NW = _NC * _NSC
_NPC = 32
_G = _NPC * _KNN
_WR = _PR // _NW
_NCH = _WR // (_NPC // 8)


def _row_of(n):
    q_lo = n & 511
    grp = (n & 4095) >> 9
    return (n >> 12) * _BN + (q_lo << 3) + grp


def _encode_body(f0, f1, ed, w0, b0, w1, b1, wg0, wg1, bg, z_out, e_out):
    i = pl.program_id(0)
    a0 = jnp.maximum(
        jnp.dot(f0[...], w0[...], preferred_element_type=jnp.float32) + b0[...], 0.0)
    a1 = jnp.maximum(
        jnp.dot(f1[...], w1[...], preferred_element_type=jnp.float32) + b1[...], 0.0)
    z = jnp.dot(a0, wg0[...], preferred_element_type=jnp.float32)
    z = z + jnp.dot(a1, wg1[...], preferred_element_type=jnp.float32)
    z = z + bg[...]
    gid = i * _BN + lax.broadcasted_iota(jnp.int32, (_BN, _KNN), 0)
    tv = jnp.where(gid < _N, ed[...], 0)
    z_out[...] = z
    for l in range(8):
        e_out[:, l * _ZW:(l + 1) * _ZW] = tv[l * 512:(l + 1) * 512, :]


def _encode(fts0, fts1, edge, W0, b0, W1, b1, Wg0, Wg1, bg):
    full = lambda r, c: pl.BlockSpec((r, c), lambda i: (0, 0))
    return pl.pallas_call(
        _encode_body,
        grid=(_NB,),
        in_specs=[
            pl.BlockSpec((_BN, _D0), lambda i: (i, 0)),
            pl.BlockSpec((_BN, _D1), lambda i: (i, 0)),
            pl.BlockSpec((_BN, _KNN), lambda i: (i, 0)),
            full(_D0, _H0), full(1, _H0),
            full(_D1, _H1), full(1, _H1),
            full(_H0, _ZW), full(_H1, _ZW), full(1, _ZW),
        ],
        out_specs=[
            pl.BlockSpec((_BN, _ZW), lambda i: (i, 0)),
            pl.BlockSpec((512, 128), lambda i: (i, 0)),
        ],
        out_shape=[
            jax.ShapeDtypeStruct((_NS_TOT, _ZW), jnp.float32),
            jax.ShapeDtypeStruct((_PR, 128), jnp.int32),
        ],
    )(fts0, fts1, edge, W0, b0, W1, b1, Wg0, Wg1, bg)


def _gather_sum_body(z_hbm, edge_hbm, out_hbm, idx_flat, buf_v, sem0, sem1):
    cc = lax.axis_index("c")
    wid = lax.axis_index("s") * _NC + cc
    base = wid * _WR
    z_tbl = z_hbm.at[cc]
    sems = (sem0, sem1)

    def stage(idx_raw):
        pltpu.sync_copy(edge_hbm.at[pl.ds(base, _WR)], idx_raw)

        def repack(r, carry):
            for s in range(8):
                idx_flat[pl.ds(r * 128 + s * _ZW, _ZW)] = \
                    idx_raw[r, s * _ZW:(s + 1) * _ZW]
            return carry
        lax.fori_loop(0, _WR, repack, 0)

    pl.run_scoped(stage, pltpu.VMEM((_WR, 128), jnp.int32))

    pltpu.async_copy(z_tbl.at[idx_flat.at[pl.ds(0, _G)]], buf_v.at[0], sem0)
    pltpu.async_copy(z_tbl.at[idx_flat.at[pl.ds(_G, _G)]], buf_v.at[1], sem1)

    def main(acc_v):
        def accumulate(p, c):
            def node(u, carry):
                vals = [buf_v[p, u * _KNN + k, :] for k in range(_KNN)]
                while len(vals) > 1:
                    nxt = [vals[i] + vals[i + 1] for i in range(0, len(vals) - 1, 2)]
                    if len(vals) % 2:
                        nxt.append(vals[-1])
                    vals = nxt
                acc_v[pl.ds((c * _NPC + u) * _ZW, _ZW)] = vals[0]
                return carry
            lax.fori_loop(0, _NPC, node, 0)

        def chunk_pair(j, carry):
            for p in range(2):
                c = 2 * j + p
                pltpu.make_async_copy(
                    z_tbl.at[pl.ds(0, _G)], buf_v.at[p], sems[p]).wait()
                accumulate(p, c)

                @pl.when(c + 2 < _NCH)
                def _start_next():
                    pltpu.async_copy(
                        z_tbl.at[idx_flat.at[pl.ds((c + 2) * _G, _G)]],
                        buf_v.at[p], sems[p])
            return carry

        lax.fori_loop(0, _NCH // 2, chunk_pair, 0)
        pltpu.sync_copy(acc_v, out_hbm.at[wid])

    pl.run_scoped(main, pltpu.VMEM((_WR * 128,), jnp.float32))


def _gather_sum(z_lin, edge_packed):
    mesh = plsc.VectorSubcoreMesh(core_axis_name="c", subcore_axis_name="s")
    return pl.kernel(
        _gather_sum_body,
        out_type=jax.ShapeDtypeStruct((_NW, _WR * 128), jnp.float32),
        name="gather_sum",
        mesh=mesh,
        compiler_params=pltpu.CompilerParams(use_tc_tiling_on_sc=False),
        scratch_types=[
            pltpu.VMEM((_WR * 128,), jnp.int32),
            pltpu.VMEM((2, _G, _ZW), jnp.float32),
            pltpu.SemaphoreType.DMA,
            pltpu.SemaphoreType.DMA,
        ],
    )(z_lin, edge_packed)


def _log_softmax_body(sp_ref, o_ref):
    sp = sp_ref[...]
    s = jnp.concatenate(
        [sp[:, l * _ZW:(l + 1) * _ZW] for l in range(8)], axis=0)
    col = lax.broadcasted_iota(jnp.int32, s.shape, 1)
    mask = col < _NCAT
    m = jnp.max(jnp.where(mask, s, -jnp.inf), axis=1, keepdims=True)
    e = jnp.where(mask, jnp.exp(s - m), 0.0)
    lse = jnp.log(jnp.sum(e, axis=1, keepdims=True))
    o_ref[...] = (s - m - lse)[:, :_NCAT]


def _log_softmax(sums_packed):
    return pl.pallas_call(
        _log_softmax_body,
        grid=(_NB,),
        in_specs=[pl.BlockSpec((512, 128), lambda i: (i, 0))],
        out_specs=pl.BlockSpec((_BN, _NCAT), lambda i: (i, 0)),
        out_shape=jax.ShapeDtypeStruct((_N, _NCAT), jnp.float32),
    )(sums_packed)


def kernel(fts0, fts1, edge_dict, W0, b0, W1, b1, Wg, bg):
    scale = jnp.float32(1.0 / _KNN)
    wg_pad = jnp.pad(Wg * scale, ((0, 0), (0, _ZW - _NCAT)))
    bg_pad = jnp.pad(bg * scale, (0, _ZW - _NCAT)).reshape(1, _ZW)
    zp, ep = _encode(fts0, fts1, edge_dict, W0, b0.reshape(1, _H0), W1,
                     b1.reshape(1, _H1), wg_pad[:_H0], wg_pad[_H0:], bg_pad)
    z2 = jnp.stack([zp, zp])
    sums_packed = _gather_sum(z2, ep).reshape(_PR, 128)
    return _log_softmax(sums_packed)


# speedup vs baseline: 1.5039x; 1.5039x over previous
import jax
import jax.numpy as jnp
from jax import lax
from jax.experimental import pallas as pl
from jax.experimental.pallas import tpu as pltpu
from jax.experimental.pallas import tpu_sc as plsc

_N = 100000
_KNN = 16
_D0, _D1 = 128, 256
_H0, _H1 = 32, 16
_NCAT = 10
_ZW = 16

_NC, _NS = 2, 16
_NW = _NC * _NS
_NPC = 32
_G = _NPC * _KNN
_R = ((_N + _NW * 2 * _NPC - 1) // (_NW * 2 * _NPC)) * (2 * _NPC)
_NCH = _R // _NPC


def _encode_body(f0, f1, w0, b0, w1, b1, wg0, wg1, bg, out):
    a0 = jnp.maximum(
        jnp.dot(f0[...], w0[...], preferred_element_type=jnp.float32) + b0[...], 0.0)
    a1 = jnp.maximum(
        jnp.dot(f1[...], w1[...], preferred_element_type=jnp.float32) + b1[...], 0.0)
    z = jnp.dot(a0, wg0[...], preferred_element_type=jnp.float32)
    z = z + jnp.dot(a1, wg1[...], preferred_element_type=jnp.float32)
    out[...] = z + bg[...]


def _encode(fts0, fts1, W0, b0, W1, b1, Wg0, Wg1, bg):
    bn = 2000
    grid = (_N // bn,)
    full = lambda r, c: pl.BlockSpec((r, c), lambda i: (0, 0))
    return pl.pallas_call(
        _encode_body,
        grid=grid,
        in_specs=[
            pl.BlockSpec((bn, _D0), lambda i: (i, 0)),
            pl.BlockSpec((bn, _D1), lambda i: (i, 0)),
            full(_D0, _H0), full(1, _H0),
            full(_D1, _H1), full(1, _H1),
            full(_H0, _ZW), full(_H1, _ZW), full(1, _ZW),
        ],
        out_specs=pl.BlockSpec((bn, _ZW), lambda i: (i, 0)),
        out_shape=jax.ShapeDtypeStruct((_N, _ZW), jnp.float32),
    )(fts0, fts1, W0, b0, W1, b1, Wg0, Wg1, bg)


def _gather_sum_body(z_hbm, edge_hbm, out_hbm, idx_flat, buf_v, sem0, sem1):
    wid = lax.axis_index("s") * _NC + lax.axis_index("c")
    base = jnp.minimum(wid * _R, _N - _R)
    sems = (sem0, sem1)

    def stage(idx_raw):
        pltpu.sync_copy(edge_hbm.at[pl.ds(base, _R)], idx_raw)

        def repack(c, carry):
            for n in range(8):
                idx_flat[pl.ds((c * 8 + n) * _KNN, _KNN)] = idx_raw[c * 8 + n, :]
            return carry
        lax.fori_loop(0, _R // 8, repack, 0)

    pl.run_scoped(stage, pltpu.VMEM((_R, _KNN), jnp.int32))

    pltpu.async_copy(z_hbm.at[idx_flat.at[pl.ds(0, _G)]], buf_v.at[0], sem0)
    pltpu.async_copy(z_hbm.at[idx_flat.at[pl.ds(_G, _G)]], buf_v.at[1], sem1)

    def main(acc_v):
        def accumulate(p, c):
            def node(n, carry):
                vals = [buf_v[p, n * _KNN + k, :] for k in range(_KNN)]
                while len(vals) > 1:
                    nxt = [vals[i] + vals[i + 1] for i in range(0, len(vals) - 1, 2)]
                    if len(vals) % 2:
                        nxt.append(vals[-1])
                    vals = nxt
                acc_v[c * _NPC + n, :] = vals[0]
                return carry
            lax.fori_loop(0, _NPC, node, 0)

        def chunk_pair(j, carry):
            for p in range(2):
                c = 2 * j + p
                pltpu.make_async_copy(
                    z_hbm.at[pl.ds(0, _G)], buf_v.at[p], sems[p]).wait()
                accumulate(p, c)

                @pl.when(c + 2 < _NCH)
                def _start_next():
                    pltpu.async_copy(
                        z_hbm.at[idx_flat.at[pl.ds((c + 2) * _G, _G)]],
                        buf_v.at[p], sems[p])
            return carry

        lax.fori_loop(0, _NCH // 2, chunk_pair, 0)
        pltpu.sync_copy(acc_v, out_hbm.at[pl.ds(base, _R)])

    pl.run_scoped(main, pltpu.VMEM((_R, _ZW), jnp.float32))


def _gather_sum(z, edge):
    mesh = plsc.VectorSubcoreMesh(core_axis_name="c", subcore_axis_name="s")
    return pl.kernel(
        _gather_sum_body,
        out_type=jax.ShapeDtypeStruct((_N, _ZW), jnp.float32),
        mesh=mesh,
        compiler_params=pltpu.CompilerParams(use_tc_tiling_on_sc=False),
        scratch_types=[
            pltpu.VMEM((_R * _KNN,), jnp.int32),
            pltpu.VMEM((2, _G, _ZW), jnp.float32),
            pltpu.SemaphoreType.DMA,
            pltpu.SemaphoreType.DMA,
        ],
    )(z, edge)


def _log_softmax_body(s_ref, o_ref):
    s = s_ref[...]
    col = lax.broadcasted_iota(jnp.int32, s.shape, 1)
    mask = col < _NCAT
    m = jnp.max(jnp.where(mask, s, -jnp.inf), axis=1, keepdims=True)
    e = jnp.where(mask, jnp.exp(s - m), 0.0)
    lse = jnp.log(jnp.sum(e, axis=1, keepdims=True))
    o_ref[...] = (s - m - lse)[:, :_NCAT]


def _log_softmax(sums):
    bc = 2000
    grid = (_N // bc,)
    return pl.pallas_call(
        _log_softmax_body,
        grid=grid,
        in_specs=[pl.BlockSpec((bc, _ZW), lambda i: (i, 0))],
        out_specs=pl.BlockSpec((bc, _NCAT), lambda i: (i, 0)),
        out_shape=jax.ShapeDtypeStruct((_N, _NCAT), jnp.float32),
    )(sums)


def kernel(fts0, fts1, edge_dict, W0, b0, W1, b1, Wg, bg):
    scale = jnp.float32(1.0 / _KNN)
    wg_pad = jnp.pad(Wg * scale, ((0, 0), (0, _ZW - _NCAT)))
    bg_pad = jnp.pad(bg * scale, (0, _ZW - _NCAT)).reshape(1, _ZW)
    z = _encode(fts0, fts1, W0, b0.reshape(1, _H0), W1, b1.reshape(1, _H1),
                wg_pad[:_H0], wg_pad[_H0:], bg_pad)
    sums = _gather_sum(z, edge_dict)
    return _log_softmax(sums)


# speedup vs baseline: 1.5374x; 1.0223x over previous
import jax
import jax.numpy as jnp
from jax import lax
from jax.experimental import pallas as pl
from jax.experimental.pallas import tpu as pltpu
from jax.experimental.pallas import tpu_sc as plsc

_N = 100000
_KNN = 16
_D0, _D1 = 128, 256
_H0, _H1 = 32, 16
_NCAT = 10
_ZW = 16

_NC, _NS = 2, 16
_NW = _NC * _NS
_NPC = 48
_G = _NPC * _KNN
_R = ((_N + _NW * 2 * _NPC - 1) // (_NW * 2 * _NPC)) * (2 * _NPC)
_NCH = _R // _NPC


def _encode_body(f0, f1, w0, b0, w1, b1, wg0, wg1, bg, out):
    a0 = jnp.maximum(
        jnp.dot(f0[...], w0[...], preferred_element_type=jnp.float32) + b0[...], 0.0)
    a1 = jnp.maximum(
        jnp.dot(f1[...], w1[...], preferred_element_type=jnp.float32) + b1[...], 0.0)
    z = jnp.dot(a0, wg0[...], preferred_element_type=jnp.float32)
    z = z + jnp.dot(a1, wg1[...], preferred_element_type=jnp.float32)
    out[...] = z + bg[...]


def _encode(fts0, fts1, W0, b0, W1, b1, Wg0, Wg1, bg):
    bn = 2000
    grid = (_N // bn,)
    full = lambda r, c: pl.BlockSpec((r, c), lambda i: (0, 0))
    return pl.pallas_call(
        _encode_body,
        grid=grid,
        in_specs=[
            pl.BlockSpec((bn, _D0), lambda i: (i, 0)),
            pl.BlockSpec((bn, _D1), lambda i: (i, 0)),
            full(_D0, _H0), full(1, _H0),
            full(_D1, _H1), full(1, _H1),
            full(_H0, _ZW), full(_H1, _ZW), full(1, _ZW),
        ],
        out_specs=pl.BlockSpec((bn, _ZW), lambda i: (i, 0)),
        out_shape=jax.ShapeDtypeStruct((_N, _ZW), jnp.float32),
    )(fts0, fts1, W0, b0, W1, b1, Wg0, Wg1, bg)


def _gather_sum_body(z_hbm, edge_hbm, out_hbm, idx_flat, buf_v, sem0, sem1):
    wid = lax.axis_index("s") * _NC + lax.axis_index("c")
    base = jnp.minimum(wid * _R, _N - _R)
    sems = (sem0, sem1)

    def stage(idx_raw):
        pltpu.sync_copy(edge_hbm.at[pl.ds(base, _R)], idx_raw)

        def repack(c, carry):
            for n in range(8):
                idx_flat[pl.ds((c * 8 + n) * _KNN, _KNN)] = idx_raw[c * 8 + n, :]
            return carry
        lax.fori_loop(0, _R // 8, repack, 0)

    pl.run_scoped(stage, pltpu.VMEM((_R, _KNN), jnp.int32))

    pltpu.async_copy(z_hbm.at[idx_flat.at[pl.ds(0, _G)]], buf_v.at[0], sem0)
    pltpu.async_copy(z_hbm.at[idx_flat.at[pl.ds(_G, _G)]], buf_v.at[1], sem1)

    def main(acc_v):
        def accumulate(p, c):
            def node(n, carry):
                vals = [buf_v[p, n * _KNN + k, :] for k in range(_KNN)]
                while len(vals) > 1:
                    nxt = [vals[i] + vals[i + 1] for i in range(0, len(vals) - 1, 2)]
                    if len(vals) % 2:
                        nxt.append(vals[-1])
                    vals = nxt
                acc_v[c * _NPC + n, :] = vals[0]
                return carry
            lax.fori_loop(0, _NPC, node, 0)

        def chunk_pair(j, carry):
            for p in range(2):
                c = 2 * j + p
                pltpu.make_async_copy(
                    z_hbm.at[pl.ds(0, _G)], buf_v.at[p], sems[p]).wait()
                accumulate(p, c)

                @pl.when(c + 2 < _NCH)
                def _start_next():
                    pltpu.async_copy(
                        z_hbm.at[idx_flat.at[pl.ds((c + 2) * _G, _G)]],
                        buf_v.at[p], sems[p])
            return carry

        lax.fori_loop(0, _NCH // 2, chunk_pair, 0)
        pltpu.sync_copy(acc_v, out_hbm.at[pl.ds(base, _R)])

    pl.run_scoped(main, pltpu.VMEM((_R, _ZW), jnp.float32))


def _gather_sum(z, edge):
    mesh = plsc.VectorSubcoreMesh(core_axis_name="c", subcore_axis_name="s")
    return pl.kernel(
        _gather_sum_body,
        out_type=jax.ShapeDtypeStruct((_N, _ZW), jnp.float32),
        mesh=mesh,
        compiler_params=pltpu.CompilerParams(use_tc_tiling_on_sc=False),
        scratch_types=[
            pltpu.VMEM((_R * _KNN,), jnp.int32),
            pltpu.VMEM((2, _G, _ZW), jnp.float32),
            pltpu.SemaphoreType.DMA,
            pltpu.SemaphoreType.DMA,
        ],
    )(z, edge)


def _log_softmax_body(s_ref, o_ref):
    s = s_ref[...]
    col = lax.broadcasted_iota(jnp.int32, s.shape, 1)
    mask = col < _NCAT
    m = jnp.max(jnp.where(mask, s, -jnp.inf), axis=1, keepdims=True)
    e = jnp.where(mask, jnp.exp(s - m), 0.0)
    lse = jnp.log(jnp.sum(e, axis=1, keepdims=True))
    o_ref[...] = (s - m - lse)[:, :_NCAT]


def _log_softmax(sums):
    bc = 2000
    grid = (_N // bc,)
    return pl.pallas_call(
        _log_softmax_body,
        grid=grid,
        in_specs=[pl.BlockSpec((bc, _ZW), lambda i: (i, 0))],
        out_specs=pl.BlockSpec((bc, _NCAT), lambda i: (i, 0)),
        out_shape=jax.ShapeDtypeStruct((_N, _NCAT), jnp.float32),
    )(sums)


def kernel(fts0, fts1, edge_dict, W0, b0, W1, b1, Wg, bg):
    scale = jnp.float32(1.0 / _KNN)
    wg_pad = jnp.pad(Wg * scale, ((0, 0), (0, _ZW - _NCAT)))
    bg_pad = jnp.pad(bg * scale, (0, _ZW - _NCAT)).reshape(1, _ZW)
    z = _encode(fts0, fts1, W0, b0.reshape(1, _H0), W1, b1.reshape(1, _H1),
                wg_pad[:_H0], wg_pad[_H0:], bg_pad)
    sums = _gather_sum(z, edge_dict)
    return _log_softmax(sums)
